# baseline (speedup 1.0000x reference)
import functools

import jax
import jax.numpy as jnp
from jax import lax
from jax.experimental import pallas as pl
from jax.experimental.pallas import tpu as pltpu
from jax.experimental.pallas import tpu_sc as plsc

N = 10000
E = 160000
HID = 64
IN_DIM = 128

NC = 2
NS = 16
NW = NC * NS
CH = 128
NCHK = 40
EPT = NCHK * CH
E_PAD = NW * EPT
NA = N + 8
RPT = 624
REM_Z = NA - NS * RPT
REM_O = N - NS * RPT
DEGW = 8

NPAD = 10240
BLK = 1024

_sc_mesh = functools.partial(
    plsc.VectorSubcoreMesh, core_axis_name="c", subcore_axis_name="s"
)
_sc_params = pltpu.CompilerParams(use_tc_tiling_on_sc=False)


def _sc_degree(dsti, zeros_nd):
    @functools.partial(
        pl.kernel,
        out_type=jax.ShapeDtypeStruct((NC, N, DEGW), jnp.float32),
        mesh=_sc_mesh(),
        compiler_params=_sc_params,
        scratch_types=[
            pltpu.VMEM_SHARED((NA, DEGW), jnp.float32),
            pltpu.VMEM((NCHK, CH), jnp.int32),
            pltpu.VMEM((CH, DEGW), jnp.float32),
            pltpu.SemaphoreType.DMA,
        ],
    )
    def k(di_hbm, z_hbm, out_hbm, acc, idst, ones_v, ssem):
        c = lax.axis_index("c")
        s = lax.axis_index("s")
        wid = c * NS + s

        @pl.loop(0, CH)
        def _(i):
            ones_v[i, :] = jnp.ones((DEGW,), jnp.float32)

        pltpu.sync_copy(di_hbm.at[wid], idst)
        pltpu.sync_copy(z_hbm.at[pl.ds(s * RPT, RPT)],
                        acc.at[pl.ds(s * RPT, RPT)])

        @pl.when(s == NS - 1)
        def _():
            pltpu.sync_copy(z_hbm.at[pl.ds(NS * RPT, REM_Z)],
                            acc.at[pl.ds(NS * RPT, REM_Z)])

        plsc.subcore_barrier()

        @pl.loop(0, NCHK)
        def _(j):
            pltpu.async_copy(ones_v, acc.at[idst.at[j]], ssem, add=True)

        @pl.loop(0, NCHK)
        def _(j):
            pltpu.make_async_copy(ones_v, acc.at[idst.at[j]], ssem).wait()

        plsc.subcore_barrier()
        pltpu.sync_copy(acc.at[pl.ds(s * RPT, RPT)],
                        out_hbm.at[c, pl.ds(s * RPT, RPT)])

        @pl.when(s == NS - 1)
        def _():
            pltpu.sync_copy(acc.at[pl.ds(NS * RPT, REM_O)],
                            out_hbm.at[c, pl.ds(NS * RPT, REM_O)])

    return k(dsti, zeros_nd)


def _sc_prop(y, srci, dsti, zeros_nh):
    @functools.partial(
        pl.kernel,
        out_type=jax.ShapeDtypeStruct((NC, N, HID), jnp.float32),
        mesh=_sc_mesh(),
        compiler_params=_sc_params,
        scratch_types=[
            pltpu.VMEM_SHARED((NA, HID), jnp.float32),
            pltpu.VMEM((NCHK, CH), jnp.int32),
            pltpu.VMEM((NCHK, CH), jnp.int32),
            pltpu.VMEM((CH, HID), jnp.float32),
        ],
    )
    def k(y_hbm, si_hbm, di_hbm, z_hbm, out_hbm,
          acc, isrc, idst, rows0):
        c = lax.axis_index("c")
        s = lax.axis_index("s")
        wid = c * NS + s

        pltpu.sync_copy(si_hbm.at[wid], isrc)
        pltpu.sync_copy(di_hbm.at[wid], idst)
        pltpu.sync_copy(z_hbm.at[pl.ds(s * RPT, RPT)],
                        acc.at[pl.ds(s * RPT, RPT)])

        @pl.when(s == NS - 1)
        def _():
            pltpu.sync_copy(z_hbm.at[pl.ds(NS * RPT, REM_Z)],
                            acc.at[pl.ds(NS * RPT, REM_Z)])

        plsc.subcore_barrier()

        @pl.loop(0, NCHK)
        def _(j):
            pltpu.sync_copy(y_hbm.at[isrc.at[j]], rows0)
            pltpu.sync_copy(rows0, acc.at[idst.at[j]], add=True)

        plsc.subcore_barrier()
        pltpu.sync_copy(acc.at[pl.ds(s * RPT, RPT)],
                        out_hbm.at[c, pl.ds(s * RPT, RPT)])

        @pl.when(s == NS - 1)
        def _():
            pltpu.sync_copy(acc.at[pl.ds(NS * RPT, REM_O)],
                            out_hbm.at[c, pl.ds(NS * RPT, REM_O)])

    return k(y, srci, dsti, zeros_nh)


def _tc_prescale(degp, h):
    def body(degp_ref, h_ref, dis_ref, y_ref):
        deg = 1.0 + degp_ref[0, :, 0:1] + degp_ref[1, :, 0:1]
        dis = lax.rsqrt(deg)
        dis_ref[...] = dis
        y_ref[...] = h_ref[...] * dis

    return pl.pallas_call(
        body,
        out_shape=(
            jax.ShapeDtypeStruct((N, 1), jnp.float32),
            jax.ShapeDtypeStruct((N, HID), jnp.float32),
        ),
    )(degp, h)


def _tc_mid(p, y1, dis, W1, b1, Ws, bs):
    def body(p_ref, y1_ref, dis_ref, w1_ref, b1_ref, ws_ref, bs_ref,
             y2_ref, hp_ref):
        dis = dis_ref[...]
        g = dis * (p_ref[0] + p_ref[1] + y1_ref[...])
        x1 = jnp.maximum(
            jnp.dot(g, w1_ref[...], preferred_element_type=jnp.float32,
                    precision=lax.Precision.HIGHEST) + b1_ref[...],
            0.0)
        y2_ref[...] = dis * x1
        h_ = jnp.dot(g, ws_ref[...], preferred_element_type=jnp.float32,
                     precision=lax.Precision.HIGHEST) + bs_ref[...]
        hp_ref[pl.ds(0, N), :] = h_
        hp_ref[pl.ds(N, NPAD - N), :] = jnp.zeros(
            (NPAD - N, IN_DIM), jnp.float32)

    return pl.pallas_call(
        body,
        out_shape=(
            jax.ShapeDtypeStruct((N, HID), jnp.float32),
            jax.ShapeDtypeStruct((NPAD, IN_DIM), jnp.float32),
        ),
    )(p, y1, dis, W1, b1, Ws, bs)


def _tc_final(q, y2, dis, W2, b2):
    def body(q_ref, y2_ref, dis_ref, w2_ref, b2_ref, out_ref):
        g2 = dis_ref[...] * (q_ref[0] + q_ref[1] + y2_ref[...])
        out_ref[...] = jnp.dot(
            g2, w2_ref[...], preferred_element_type=jnp.float32,
            precision=lax.Precision.HIGHEST) + b2_ref[...]

    return pl.pallas_call(
        body,
        out_shape=jax.ShapeDtypeStruct((N, IN_DIM), jnp.float32),
    )(q, y2, dis, W2, b2)


def _tc_adj(hp):
    def body(a_ref, b_ref, out_ref):
        a = a_ref[...].astype(jnp.bfloat16)
        b = b_ref[...].astype(jnp.bfloat16)
        out_ref[...] = lax.dot_general(
            a, b, (((1,), (1,)), ((), ())),
            preferred_element_type=jnp.float32)

    nblk = NPAD // BLK
    return pl.pallas_call(
        body,
        grid=(nblk, nblk),
        in_specs=[
            pl.BlockSpec((BLK, IN_DIM), lambda i, j: (i, 0)),
            pl.BlockSpec((BLK, IN_DIM), lambda i, j: (j, 0)),
        ],
        out_specs=pl.BlockSpec((BLK, BLK), lambda i, j: (i, j)),
        out_shape=jax.ShapeDtypeStruct((N, N), jnp.float32),
    )(hp, hp)


def kernel(h, edge_index, W1, b1, W2, b2, Ws, bs):
    src = edge_index[0].astype(jnp.int32)
    dst = edge_index[1].astype(jnp.int32)
    ppw = EPT - E // NW
    pad_src = jnp.zeros((NW, ppw), jnp.int32)
    pad_dst = jnp.broadcast_to(
        N + (jnp.arange(ppw, dtype=jnp.int32) % 8), (NW, ppw))
    srci = jnp.concatenate(
        [src.reshape(NW, E // NW), pad_src], axis=1).reshape(NW, NCHK, CH)
    dsti = jnp.concatenate(
        [dst.reshape(NW, E // NW), pad_dst], axis=1).reshape(NW, NCHK, CH)
    b1r = b1.reshape(1, HID)
    b2r = b2.reshape(1, IN_DIM)
    bsr = bs.reshape(1, IN_DIM)
    zeros_nd = jnp.zeros((NA, DEGW), jnp.float32)
    zeros_nh = jnp.zeros((NA, HID), jnp.float32)

    degp = _sc_degree(dsti, zeros_nd)
    dis, y1 = _tc_prescale(degp, h)
    p = _sc_prop(y1, srci, dsti, zeros_nh)
    y2, hp = _tc_mid(p, y1, dis, W1, b1r, Ws, bsr)
    q = _sc_prop(y2, srci, dsti, zeros_nh)
    x2 = _tc_final(q, y2, dis, W2, b2r)
    adj = _tc_adj(hp)
    return (x2, adj)


# speedup vs baseline: 1.3423x; 1.3423x over previous
import functools

import jax
import jax.numpy as jnp
from jax import lax
from jax.experimental import pallas as pl
from jax.experimental.pallas import tpu as pltpu
from jax.experimental.pallas import tpu_sc as plsc

N = 10000
E = 160000
HID = 64
IN_DIM = 128

NC = 2
NS = 16
NW = NC * NS
EPT = E // NW
CH = 128
NFULL = EPT // CH
TAIL = EPT - NFULL * CH
RPT = 624
REM_O = N - NS * RPT
NA = N + 8
REM_Z = NA - NS * RPT
NCHK = 40
EPTD = NCHK * CH
DEGW = 8

NPAD = 10240
BLK = 2048

_sc_mesh = functools.partial(
    plsc.VectorSubcoreMesh, core_axis_name="c", subcore_axis_name="s"
)
_sc_params = pltpu.CompilerParams(use_tc_tiling_on_sc=False)


def _sc_degree(dsti, zeros_nd):
    @functools.partial(
        pl.kernel,
        out_type=jax.ShapeDtypeStruct((NC, N, DEGW), jnp.float32),
        mesh=_sc_mesh(),
        compiler_params=_sc_params,
        scratch_types=[
            pltpu.VMEM_SHARED((NA, DEGW), jnp.float32),
            pltpu.VMEM((NCHK, CH), jnp.int32),
            pltpu.VMEM((CH, DEGW), jnp.float32),
            pltpu.SemaphoreType.DMA,
        ],
    )
    def k(di_hbm, z_hbm, out_hbm, acc, idst, ones_v, ssem):
        c = lax.axis_index("c")
        s = lax.axis_index("s")
        wid = c * NS + s

        @pl.loop(0, CH)
        def _(i):
            ones_v[i, :] = jnp.ones((DEGW,), jnp.float32)

        pltpu.sync_copy(di_hbm.at[wid], idst)
        pltpu.sync_copy(z_hbm.at[pl.ds(s * RPT, RPT)],
                        acc.at[pl.ds(s * RPT, RPT)])

        @pl.when(s == NS - 1)
        def _():
            pltpu.sync_copy(z_hbm.at[pl.ds(NS * RPT, REM_Z)],
                            acc.at[pl.ds(NS * RPT, REM_Z)])

        plsc.subcore_barrier()

        @pl.loop(0, NCHK)
        def _(j):
            pltpu.async_copy(ones_v, acc.at[idst.at[j]], ssem, add=True)

        @pl.loop(0, NCHK)
        def _(j):
            pltpu.make_async_copy(ones_v, acc.at[idst.at[j]], ssem).wait()

        plsc.subcore_barrier()
        pltpu.sync_copy(acc.at[pl.ds(s * RPT, RPT)],
                        out_hbm.at[c, pl.ds(s * RPT, RPT)])

        @pl.when(s == NS - 1)
        def _():
            pltpu.sync_copy(acc.at[pl.ds(NS * RPT, REM_O)],
                            out_hbm.at[c, pl.ds(NS * RPT, REM_O)])

    return k(dsti, zeros_nd)


def _sc_prop(y, src_i32, dst_i32, zeros_nh):
    @functools.partial(
        pl.kernel,
        out_type=jax.ShapeDtypeStruct((NC, N, HID), jnp.float32),
        mesh=_sc_mesh(),
        compiler_params=_sc_params,
        scratch_types=[
            pltpu.VMEM_SHARED((N, HID), jnp.float32),
            pltpu.VMEM((CH,), jnp.int32),
            pltpu.VMEM((CH,), jnp.int32),
            pltpu.VMEM((CH, HID), jnp.float32),
            pltpu.VMEM((TAIL,), jnp.int32),
            pltpu.VMEM((TAIL,), jnp.int32),
            pltpu.VMEM((TAIL, HID), jnp.float32),
        ],
    )
    def k(y_hbm, src_hbm, dst_hbm, z_hbm, out_hbm,
          acc, isrc, idst, rows, isrc_t, idst_t, rows_t):
        c = lax.axis_index("c")
        s = lax.axis_index("s")
        wid = c * NS + s

        pltpu.sync_copy(z_hbm.at[pl.ds(s * RPT, RPT)],
                        acc.at[pl.ds(s * RPT, RPT)])

        @pl.when(s == NS - 1)
        def _():
            pltpu.sync_copy(z_hbm.at[pl.ds(NS * RPT, REM_O)],
                            acc.at[pl.ds(NS * RPT, REM_O)])

        plsc.subcore_barrier()

        base = wid * EPT

        @pl.loop(0, NFULL)
        def _(j):
            off = base + j * CH
            pltpu.sync_copy(src_hbm.at[pl.ds(off, CH)], isrc)
            pltpu.sync_copy(dst_hbm.at[pl.ds(off, CH)], idst)
            pltpu.sync_copy(y_hbm.at[isrc], rows)
            pltpu.sync_copy(rows, acc.at[idst], add=True)

        off_t = base + NFULL * CH
        pltpu.sync_copy(src_hbm.at[pl.ds(off_t, TAIL)], isrc_t)
        pltpu.sync_copy(dst_hbm.at[pl.ds(off_t, TAIL)], idst_t)
        pltpu.sync_copy(y_hbm.at[isrc_t], rows_t)
        pltpu.sync_copy(rows_t, acc.at[idst_t], add=True)

        plsc.subcore_barrier()
        pltpu.sync_copy(acc.at[pl.ds(s * RPT, RPT)],
                        out_hbm.at[c, pl.ds(s * RPT, RPT)])

        @pl.when(s == NS - 1)
        def _():
            pltpu.sync_copy(acc.at[pl.ds(NS * RPT, REM_O)],
                            out_hbm.at[c, pl.ds(NS * RPT, REM_O)])

    return k(y, src_i32, dst_i32, zeros_nh)


def _tc_prescale(degp, h):
    def body(degp_ref, h_ref, dis_ref, y_ref):
        deg = 1.0 + degp_ref[0, :, 0:1] + degp_ref[1, :, 0:1]
        dis = lax.rsqrt(deg)
        dis_ref[...] = dis
        y_ref[...] = h_ref[...] * dis

    return pl.pallas_call(
        body,
        out_shape=(
            jax.ShapeDtypeStruct((N, 1), jnp.float32),
            jax.ShapeDtypeStruct((N, HID), jnp.float32),
        ),
    )(degp, h)


def _tc_mid(p, y1, dis, W1, b1, Ws, bs):
    def body(p_ref, y1_ref, dis_ref, w1_ref, b1_ref, ws_ref, bs_ref,
             y2_ref, hp_ref):
        dis = dis_ref[...]
        g = dis * (p_ref[0] + p_ref[1] + y1_ref[...])
        x1 = jnp.maximum(
            jnp.dot(g, w1_ref[...], preferred_element_type=jnp.float32,
                    precision=lax.Precision.HIGHEST) + b1_ref[...],
            0.0)
        y2_ref[...] = dis * x1
        h_ = jnp.dot(g, ws_ref[...], preferred_element_type=jnp.float32,
                     precision=lax.Precision.HIGHEST) + bs_ref[...]
        hp_ref[pl.ds(0, N), :] = h_
        hp_ref[pl.ds(N, NPAD - N), :] = jnp.zeros(
            (NPAD - N, IN_DIM), jnp.float32)

    return pl.pallas_call(
        body,
        out_shape=(
            jax.ShapeDtypeStruct((N, HID), jnp.float32),
            jax.ShapeDtypeStruct((NPAD, IN_DIM), jnp.float32),
        ),
    )(p, y1, dis, W1, b1, Ws, bs)


def _tc_final(q, y2, dis, W2, b2):
    def body(q_ref, y2_ref, dis_ref, w2_ref, b2_ref, out_ref):
        g2 = dis_ref[...] * (q_ref[0] + q_ref[1] + y2_ref[...])
        out_ref[...] = jnp.dot(
            g2, w2_ref[...], preferred_element_type=jnp.float32,
            precision=lax.Precision.HIGHEST) + b2_ref[...]

    return pl.pallas_call(
        body,
        out_shape=jax.ShapeDtypeStruct((N, IN_DIM), jnp.float32),
    )(q, y2, dis, W2, b2)


def _tc_adj(hp):
    def body(a_ref, b_ref, out_ref):
        a = a_ref[...].astype(jnp.bfloat16)
        b = b_ref[...].astype(jnp.bfloat16)
        out_ref[...] = lax.dot_general(
            a, b, (((1,), (1,)), ((), ())),
            preferred_element_type=jnp.float32)

    nblk = NPAD // BLK
    return pl.pallas_call(
        body,
        grid=(nblk, nblk),
        in_specs=[
            pl.BlockSpec((BLK, IN_DIM), lambda i, j: (i, 0)),
            pl.BlockSpec((BLK, IN_DIM), lambda i, j: (j, 0)),
        ],
        out_specs=pl.BlockSpec((BLK, BLK), lambda i, j: (i, j)),
        out_shape=jax.ShapeDtypeStruct((N, N), jnp.float32),
    )(hp, hp)


def kernel(h, edge_index, W1, b1, W2, b2, Ws, bs):
    src = edge_index[0].astype(jnp.int32)
    dst = edge_index[1].astype(jnp.int32)
    ppw = EPTD - E // NW
    pad_dst = jnp.broadcast_to(
        N + (jnp.arange(ppw, dtype=jnp.int32) % 8), (NW, ppw))
    dsti = jnp.concatenate(
        [dst.reshape(NW, E // NW), pad_dst], axis=1).reshape(NW, NCHK, CH)
    b1r = b1.reshape(1, HID)
    b2r = b2.reshape(1, IN_DIM)
    bsr = bs.reshape(1, IN_DIM)
    zeros_nd = jnp.zeros((NA, DEGW), jnp.float32)
    zeros_nh = jnp.zeros((N, HID), jnp.float32)

    degp = _sc_degree(dsti, zeros_nd)
    dis, y1 = _tc_prescale(degp, h)
    p = _sc_prop(y1, src, dst, zeros_nh)
    y2, hp = _tc_mid(p, y1, dis, W1, b1r, Ws, bsr)
    q = _sc_prop(y2, src, dst, zeros_nh)
    x2 = _tc_final(q, y2, dis, W2, b2r)
    adj = _tc_adj(hp)
    return (x2, adj)
